# baseline (speedup 1.0000x reference)
import jax
import jax.numpy as jnp
from jax import lax
from jax.experimental import pallas as pl
from jax.experimental.pallas import tpu as pltpu

B = 8
N = 2048
DIM = 192
S = 256
K = 32
NUM_HEADS = 6
MLP_HIDDEN = 768
HEAD_DIM = DIM // NUM_HEADS
SCALE = HEAD_DIM ** -0.5
RBLK = 256
NBLK = N // RBLK
BIGF = 3.0e38
BIGI = 2 ** 30


def _t(v):
    one = jnp.ones((1, 1), jnp.float32)
    if v.shape[0] == 1:
        return lax.dot_general(v, one, (((0,), (0,)), ((), ())),
                               preferred_element_type=jnp.float32,
                               precision=lax.Precision.HIGHEST)
    return lax.dot_general(one, v, (((1,), (1,)), ((), ())),
                           preferred_element_type=jnp.float32,
                           precision=lax.Precision.HIGHEST)


def _ln_into(x_ref, mu_ref, var_ref, w_ref, b_ref, xn_s):
    xn_s[...] = ((x_ref[0] - mu_ref[0]) / jnp.sqrt(var_ref[0] + 1e-5)
                 * w_ref[...][None, :] + b_ref[...][None, :])


def _near_kernel(x_ref, mu_ref, var_ref, n1w_ref, n1b_ref,
                 sqr_ref, sqc_ref, sw_ref, sb_ref,
                 near_ref, dmax_ref, ts_ref, tw_ref, xn_s):
    _ln_into(x_ref, mu_ref, var_ref, n1w_ref, n1b_ref, xn_s)
    sq_col = sqc_ref[0]

    ts_col = lax.dot_general(sw_ref[...], xn_s[...], (((0,), (1,)), ((), ())),
                             preferred_element_type=jnp.float32) + sb_ref[0]
    ts_ref[0] = ts_col
    tw_ref[0] = jnp.exp(ts_col)

    def blk(i, dmax):
        rb = pl.ds(i * RBLK, RBLK)
        sqb = sqr_ref[0, rb, :]
        g = lax.dot_general(xn_s[rb, :], xn_s[...], (((1,), (1,)), ((), ())),
                            preferred_element_type=jnp.float32)
        d2 = jnp.maximum(sqb + sq_col - 2.0 * g, 0.0)
        dmb = jnp.sqrt(d2) / (DIM ** 0.5)
        dmax = jnp.maximum(dmax, jnp.max(dmb))
        pos = lax.broadcasted_iota(jnp.int32, (1, K), 1)
        prev = jnp.full((RBLK, 1), -1.0, jnp.float32)
        cum = jnp.zeros((RBLK, 1), jnp.int32)
        near = jnp.zeros((RBLK, K), jnp.float32)
        for k in range(K + 1):
            c = jnp.sum((dmb == prev).astype(jnp.int32), axis=1, keepdims=True)
            newcum = cum + c
            ind = (pos >= cum) & (pos < newcum)
            near = near + jnp.where(ind, prev, 0.0)
            cum = newcum
            if k < K:
                prev = jnp.min(jnp.where(dmb > prev, dmb, BIGF), axis=1,
                               keepdims=True)
        near_ref[0, rb, :] = near
        return dmax

    dmax = lax.fori_loop(0, NBLK, blk, jnp.float32(0.0))
    dmax_ref[0, 0, :] = jnp.full((128,), dmax, jnp.float32)


def _assign_kernel(x_ref, mu_ref, var_ref, n1w_ref, n1b_ref,
                   sqr_ref, sqc_ref, densr_ref, densc_ref, dmax_ref,
                   idx_ref, xn_s, dm_s, dist_s, rank_s):
    _ln_into(x_ref, mu_ref, var_ref, n1w_ref, n1b_ref, xn_s)
    sq_col = sqc_ref[0]
    dens_col = densc_ref[0]
    dmax = dmax_ref[0, 0, 0]

    def blk(i, _):
        rb = pl.ds(i * RBLK, RBLK)
        sqb = sqr_ref[0, rb, :]
        g = lax.dot_general(xn_s[rb, :], xn_s[...], (((1,), (1,)), ((), ())),
                            preferred_element_type=jnp.float32)
        d2 = jnp.maximum(sqb + sq_col - 2.0 * g, 0.0)
        dmb = jnp.sqrt(d2) / (DIM ** 0.5)
        dm_s[rb, :] = dmb
        di = densr_ref[0, rb, :]
        masked = jnp.where(dens_col > di, dmb, dmax)
        dist_s[rb, :] = jnp.min(masked, axis=1, keepdims=True)
        return 0

    lax.fori_loop(0, NBLK, blk, 0)

    score_row = dist_s[...] * densr_ref[0]
    score_col = _t(score_row)
    jcol = lax.broadcasted_iota(jnp.int32, (1, N), 1)

    def rank_blk(i, _):
        rb = pl.ds(i * RBLK, RBLK)
        si = dist_s[rb, :] * densr_ref[0, rb, :]
        irow = lax.broadcasted_iota(jnp.int32, (RBLK, 1), 0) + i * RBLK
        beat = (score_col > si) | ((score_col == si) & (jcol < irow))
        rank_s[rb, :] = jnp.sum(beat.astype(jnp.int32), axis=1, keepdims=True)
        return 0

    lax.fori_loop(0, NBLK, rank_blk, 0)

    def colmin_blk(i, m):
        rb = pl.ds(i * RBLK, RBLK)
        is_c = rank_s[rb, :] < S
        return jnp.minimum(m, jnp.min(jnp.where(is_c, dm_s[rb, :], BIGF),
                                      axis=0, keepdims=True))

    m_col = lax.fori_loop(0, NBLK, colmin_blk, jnp.full((1, N), BIGF, jnp.float32))

    def argmin_blk(i, a):
        rb = pl.ds(i * RBLK, RBLK)
        rk = rank_s[rb, :]
        is_c = rk < S
        cand = jnp.where(is_c & (dm_s[rb, :] == m_col), rk, BIGI)
        return jnp.minimum(a, jnp.min(cand, axis=0, keepdims=True))

    idx_col = lax.fori_loop(0, NBLK, argmin_blk, jnp.full((1, N), BIGI, jnp.int32))

    rank_col = _t(rank_s[...].astype(jnp.float32)).astype(jnp.int32)
    idx_ref[0] = jnp.where(rank_col < S, rank_col, idx_col)


def _block_kernel(x_ref, mu_ref, var_ref, n1w_ref, n1b_ref,
                  idx_ref, ts_ref, tw_ref,
                  wq_ref, bq_ref, wkv_ref, bkv_ref, wp_ref, bp_ref,
                  n2w_ref, n2b_ref, w1_ref, b1_ref, w2_ref, b2_ref,
                  out_ref):
    xn = ((x_ref[0] - mu_ref[0]) / jnp.sqrt(var_ref[0] + 1e-5)
          * n1w_ref[...][None, :] + n1b_ref[...][None, :])
    ts_col = ts_ref[0]
    tw_col = tw_ref[0]
    idx_col = idx_ref[0]

    srow = lax.broadcasted_iota(jnp.int32, (S, 1), 0)
    onehot = (idx_col == srow).astype(jnp.float32)
    tw_row = _t(tw_col)
    wsum = lax.dot_general(onehot, tw_row, (((1,), (0,)), ((), ())),
                           preferred_element_type=jnp.float32,
                           precision=lax.Precision.HIGHEST)
    wf = xn * tw_row
    fsum = lax.dot_general(onehot, wf, (((1,), (0,)), ((), ())),
                           preferred_element_type=jnp.float32,
                           precision=lax.Precision.HIGHEST)
    q_in = fsum / (wsum + 1e-6)

    q = lax.dot_general(q_in, wq_ref[...], (((1,), (0,)), ((), ())),
                        preferred_element_type=jnp.float32) + bq_ref[...][None, :]
    kv = lax.dot_general(xn, wkv_ref[...], (((1,), (0,)), ((), ())),
                         preferred_element_type=jnp.float32) + bkv_ref[...][None, :]

    heads = []
    for h in range(NUM_HEADS):
        qh = q[:, h * HEAD_DIM:(h + 1) * HEAD_DIM]
        kh = kv[:, h * HEAD_DIM:(h + 1) * HEAD_DIM]
        vh = kv[:, DIM + h * HEAD_DIM:DIM + (h + 1) * HEAD_DIM]
        logits = lax.dot_general(qh, kh, (((1,), (1,)), ((), ())),
                                 preferred_element_type=jnp.float32) * SCALE
        logits = logits + ts_col
        mx = jnp.max(logits, axis=1, keepdims=True)
        p = jnp.exp(logits - mx)
        a = p / jnp.sum(p, axis=1, keepdims=True)
        heads.append(lax.dot_general(a, vh, (((1,), (0,)), ((), ())),
                                     preferred_element_type=jnp.float32))
    o = jnp.concatenate(heads, axis=1)
    attn = lax.dot_general(o, wp_ref[...], (((1,), (0,)), ((), ())),
                           preferred_element_type=jnp.float32) + bp_ref[...][None, :]
    feat = q_in + attn

    mu = jnp.mean(feat, axis=1, keepdims=True)
    var = jnp.mean((feat - mu) ** 2, axis=1, keepdims=True)
    hh = (feat - mu) / jnp.sqrt(var + 1e-5) * n2w_ref[...][None, :] + n2b_ref[...][None, :]
    h1 = lax.dot_general(hh, w1_ref[...], (((1,), (0,)), ((), ())),
                         preferred_element_type=jnp.float32) + b1_ref[...][None, :]
    h1 = 0.5 * h1 * (1.0 + lax.erf(h1 * (2.0 ** -0.5)))
    h2 = lax.dot_general(h1, w2_ref[...], (((1,), (0,)), ((), ())),
                         preferred_element_type=jnp.float32) + b2_ref[...][None, :]
    out_ref[0] = feat + h2


def _full(shape):
    return pl.BlockSpec(shape, lambda b: (0,) * len(shape))


def _batched(shape):
    return pl.BlockSpec((1,) + shape, lambda b: (b,) + (0,) * len(shape))


@jax.jit
def kernel(x, norm1_w, norm1_b, score_w, score_b, Wq, bq, Wkv, bkv, Wproj,
           bproj, norm2_w, norm2_b, W1, b1, W2, b2):
    mu = jnp.mean(x, axis=-1, keepdims=True)
    var = jnp.mean((x - mu) ** 2, axis=-1, keepdims=True)
    xn = (x - mu) / jnp.sqrt(var + 1e-5) * norm1_w + norm1_b
    sq = jnp.sum(xn * xn, axis=-1)
    sq_row = sq.reshape(B, N, 1)
    sq_col = sq.reshape(B, 1, N)

    near32, dmax, ts, tw = pl.pallas_call(
        _near_kernel,
        grid=(B,),
        in_specs=[
            _batched((N, DIM)),
            _batched((N, 1)),
            _batched((N, 1)),
            _full((DIM,)), _full((DIM,)),
            _batched((N, 1)),
            _batched((1, N)),
            _full((DIM, 1)), _full((1,)),
        ],
        out_specs=[
            _batched((N, K)),
            _batched((1, 128)),
            _batched((1, N)),
            _batched((1, N)),
        ],
        out_shape=[
            jax.ShapeDtypeStruct((B, N, K), jnp.float32),
            jax.ShapeDtypeStruct((B, 1, 128), jnp.float32),
            jax.ShapeDtypeStruct((B, 1, N), jnp.float32),
            jax.ShapeDtypeStruct((B, 1, N), jnp.float32),
        ],
        scratch_shapes=[pltpu.VMEM((N, DIM), jnp.float32)],
    )(x, mu, var, norm1_w, norm1_b, sq_row, sq_col, score_w, score_b)

    density = jnp.exp(-jnp.mean(near32 ** 2, axis=-1))
    noise = jax.random.uniform(jax.random.key(42), density.shape,
                               dtype=density.dtype) * 1e-6
    density = density + noise
    dens_row = density.reshape(B, N, 1)
    dens_col = density.reshape(B, 1, N)

    idx = pl.pallas_call(
        _assign_kernel,
        grid=(B,),
        in_specs=[
            _batched((N, DIM)),
            _batched((N, 1)),
            _batched((N, 1)),
            _full((DIM,)), _full((DIM,)),
            _batched((N, 1)),
            _batched((1, N)),
            _batched((N, 1)),
            _batched((1, N)),
            _batched((1, 128)),
        ],
        out_specs=_batched((1, N)),
        out_shape=jax.ShapeDtypeStruct((B, 1, N), jnp.int32),
        scratch_shapes=[
            pltpu.VMEM((N, DIM), jnp.float32),
            pltpu.VMEM((N, N), jnp.float32),
            pltpu.VMEM((N, 1), jnp.float32),
            pltpu.VMEM((N, 1), jnp.int32),
        ],
    )(x, mu, var, norm1_w, norm1_b, sq_row, sq_col, dens_row, dens_col, dmax)

    out = pl.pallas_call(
        _block_kernel,
        grid=(B,),
        in_specs=[
            _batched((N, DIM)),
            _batched((N, 1)),
            _batched((N, 1)),
            _full((DIM,)), _full((DIM,)),
            _batched((1, N)),
            _batched((1, N)),
            _batched((1, N)),
            _full((DIM, DIM)), _full((DIM,)),
            _full((DIM, 2 * DIM)), _full((2 * DIM,)),
            _full((DIM, DIM)), _full((DIM,)),
            _full((DIM,)), _full((DIM,)),
            _full((DIM, MLP_HIDDEN)), _full((MLP_HIDDEN,)),
            _full((MLP_HIDDEN, DIM)), _full((DIM,)),
        ],
        out_specs=_batched((S, DIM)),
        out_shape=jax.ShapeDtypeStruct((B, S, DIM), jnp.float32),
    )(x, mu, var, norm1_w, norm1_b, idx, ts, tw, Wq, bq, Wkv, bkv, Wproj, bproj,
      norm2_w, norm2_b, W1, b1, W2, b2)
    return out


# speedup vs baseline: 1.0275x; 1.0275x over previous
import jax
import jax.numpy as jnp
from jax import lax
from jax.experimental import pallas as pl
from jax.experimental.pallas import tpu as pltpu

B = 8
N = 2048
DIM = 192
S = 256
K = 32
NUM_HEADS = 6
MLP_HIDDEN = 768
HEAD_DIM = DIM // NUM_HEADS
SCALE = HEAD_DIM ** -0.5
RBLK = 512
NBLK = N // RBLK
BIGF = 3.0e38
BIGI = 2 ** 30


def _t(v):
    one = jnp.ones((1, 1), jnp.float32)
    if v.shape[0] == 1:
        return lax.dot_general(v, one, (((0,), (0,)), ((), ())),
                               preferred_element_type=jnp.float32,
                               precision=lax.Precision.HIGHEST)
    return lax.dot_general(one, v, (((1,), (1,)), ((), ())),
                           preferred_element_type=jnp.float32,
                           precision=lax.Precision.HIGHEST)


def _ln_into(x_ref, mu_ref, var_ref, w_ref, b_ref, xn_s):
    xn_s[...] = ((x_ref[0] - mu_ref[0]) / jnp.sqrt(var_ref[0] + 1e-5)
                 * w_ref[...][None, :] + b_ref[...][None, :])


def _near_kernel(x_ref, mu_ref, var_ref, n1w_ref, n1b_ref,
                 sqr_ref, sqc_ref, sw_ref, sb_ref,
                 near_ref, dmax_ref, ts_ref, tw_ref, xn_s):
    _ln_into(x_ref, mu_ref, var_ref, n1w_ref, n1b_ref, xn_s)
    sq_col = sqc_ref[0]

    ts_col = lax.dot_general(sw_ref[...], xn_s[...], (((0,), (1,)), ((), ())),
                             preferred_element_type=jnp.float32) + sb_ref[0]
    ts_ref[0] = ts_col
    tw_ref[0] = jnp.exp(ts_col)

    def blk(i, dmax):
        rb = pl.ds(i * RBLK, RBLK)
        sqb = sqr_ref[0, rb, :]
        g = lax.dot_general(xn_s[rb, :], xn_s[...], (((1,), (1,)), ((), ())),
                            preferred_element_type=jnp.float32)
        d2 = jnp.maximum(sqb + sq_col - 2.0 * g, 0.0)
        dmb = jnp.sqrt(d2) / (DIM ** 0.5)
        dmax = jnp.maximum(dmax, jnp.max(dmb))
        pos = lax.broadcasted_iota(jnp.int32, (1, K), 1)
        prev = jnp.full((RBLK, 1), -1.0, jnp.float32)
        cum = jnp.zeros((RBLK, 1), jnp.int32)
        near = jnp.zeros((RBLK, K), jnp.float32)
        for k in range(K):
            v = jnp.min(jnp.where(dmb > prev, dmb, BIGF), axis=1, keepdims=True)
            c = jnp.sum((dmb == v).astype(jnp.int32), axis=1, keepdims=True)
            newcum = cum + c
            ind = (pos >= cum) & (pos < newcum)
            near = near + jnp.where(ind, v, 0.0)
            cum = newcum
            prev = v
        near_ref[0, rb, :] = near
        return dmax

    dmax = lax.fori_loop(0, NBLK, blk, jnp.float32(0.0))
    dmax_ref[0, 0, :] = jnp.full((128,), dmax, jnp.float32)


def _assign_kernel(x_ref, mu_ref, var_ref, n1w_ref, n1b_ref,
                   sqr_ref, sqc_ref, densr_ref, densc_ref, dmax_ref,
                   idx_ref, xn_s, dm_s, dist_s, rank_s):
    _ln_into(x_ref, mu_ref, var_ref, n1w_ref, n1b_ref, xn_s)
    sq_col = sqc_ref[0]
    dens_col = densc_ref[0]
    dmax = dmax_ref[0, 0, 0]

    def blk(i, _):
        rb = pl.ds(i * RBLK, RBLK)
        sqb = sqr_ref[0, rb, :]
        g = lax.dot_general(xn_s[rb, :], xn_s[...], (((1,), (1,)), ((), ())),
                            preferred_element_type=jnp.float32)
        d2 = jnp.maximum(sqb + sq_col - 2.0 * g, 0.0)
        dmb = jnp.sqrt(d2) / (DIM ** 0.5)
        dm_s[rb, :] = dmb
        di = densr_ref[0, rb, :]
        masked = jnp.where(dens_col > di, dmb, dmax)
        dist_s[rb, :] = jnp.min(masked, axis=1, keepdims=True)
        return 0

    lax.fori_loop(0, NBLK, blk, 0)

    score_row = dist_s[...] * densr_ref[0]
    score_col = _t(score_row)
    jcol = lax.broadcasted_iota(jnp.int32, (1, N), 1)

    def rank_blk(i, _):
        rb = pl.ds(i * RBLK, RBLK)
        si = dist_s[rb, :] * densr_ref[0, rb, :]
        irow = lax.broadcasted_iota(jnp.int32, (RBLK, 1), 0) + i * RBLK
        beat = (score_col > si) | ((score_col == si) & (jcol < irow))
        rank_s[rb, :] = jnp.sum(beat.astype(jnp.int32), axis=1, keepdims=True)
        return 0

    lax.fori_loop(0, NBLK, rank_blk, 0)

    def colmin_blk(i, m):
        rb = pl.ds(i * RBLK, RBLK)
        is_c = rank_s[rb, :] < S
        return jnp.minimum(m, jnp.min(jnp.where(is_c, dm_s[rb, :], BIGF),
                                      axis=0, keepdims=True))

    m_col = lax.fori_loop(0, NBLK, colmin_blk, jnp.full((1, N), BIGF, jnp.float32))

    def argmin_blk(i, a):
        rb = pl.ds(i * RBLK, RBLK)
        rk = rank_s[rb, :]
        is_c = rk < S
        cand = jnp.where(is_c & (dm_s[rb, :] == m_col), rk, BIGI)
        return jnp.minimum(a, jnp.min(cand, axis=0, keepdims=True))

    idx_col = lax.fori_loop(0, NBLK, argmin_blk, jnp.full((1, N), BIGI, jnp.int32))

    rank_col = _t(rank_s[...].astype(jnp.float32)).astype(jnp.int32)
    idx_ref[0] = jnp.where(rank_col < S, rank_col, idx_col)


def _block_kernel(x_ref, mu_ref, var_ref, n1w_ref, n1b_ref,
                  idx_ref, ts_ref, tw_ref,
                  wq_ref, bq_ref, wkv_ref, bkv_ref, wp_ref, bp_ref,
                  n2w_ref, n2b_ref, w1_ref, b1_ref, w2_ref, b2_ref,
                  out_ref):
    xn = ((x_ref[0] - mu_ref[0]) / jnp.sqrt(var_ref[0] + 1e-5)
          * n1w_ref[...][None, :] + n1b_ref[...][None, :])
    ts_col = ts_ref[0]
    tw_col = tw_ref[0]
    idx_col = idx_ref[0]

    srow = lax.broadcasted_iota(jnp.int32, (S, 1), 0)
    onehot = (idx_col == srow).astype(jnp.float32)
    tw_row = _t(tw_col)
    wsum = lax.dot_general(onehot, tw_row, (((1,), (0,)), ((), ())),
                           preferred_element_type=jnp.float32,
                           precision=lax.Precision.HIGHEST)
    wf = xn * tw_row
    fsum = lax.dot_general(onehot, wf, (((1,), (0,)), ((), ())),
                           preferred_element_type=jnp.float32,
                           precision=lax.Precision.HIGHEST)
    q_in = fsum / (wsum + 1e-6)

    q = lax.dot_general(q_in, wq_ref[...], (((1,), (0,)), ((), ())),
                        preferred_element_type=jnp.float32) + bq_ref[...][None, :]
    kv = lax.dot_general(xn, wkv_ref[...], (((1,), (0,)), ((), ())),
                         preferred_element_type=jnp.float32) + bkv_ref[...][None, :]

    heads = []
    for h in range(NUM_HEADS):
        qh = q[:, h * HEAD_DIM:(h + 1) * HEAD_DIM]
        kh = kv[:, h * HEAD_DIM:(h + 1) * HEAD_DIM]
        vh = kv[:, DIM + h * HEAD_DIM:DIM + (h + 1) * HEAD_DIM]
        logits = lax.dot_general(qh, kh, (((1,), (1,)), ((), ())),
                                 preferred_element_type=jnp.float32) * SCALE
        logits = logits + ts_col
        mx = jnp.max(logits, axis=1, keepdims=True)
        p = jnp.exp(logits - mx)
        a = p / jnp.sum(p, axis=1, keepdims=True)
        heads.append(lax.dot_general(a, vh, (((1,), (0,)), ((), ())),
                                     preferred_element_type=jnp.float32))
    o = jnp.concatenate(heads, axis=1)
    attn = lax.dot_general(o, wp_ref[...], (((1,), (0,)), ((), ())),
                           preferred_element_type=jnp.float32) + bp_ref[...][None, :]
    feat = q_in + attn

    mu = jnp.mean(feat, axis=1, keepdims=True)
    var = jnp.mean((feat - mu) ** 2, axis=1, keepdims=True)
    hh = (feat - mu) / jnp.sqrt(var + 1e-5) * n2w_ref[...][None, :] + n2b_ref[...][None, :]
    h1 = lax.dot_general(hh, w1_ref[...], (((1,), (0,)), ((), ())),
                         preferred_element_type=jnp.float32) + b1_ref[...][None, :]
    h1 = 0.5 * h1 * (1.0 + lax.erf(h1 * (2.0 ** -0.5)))
    h2 = lax.dot_general(h1, w2_ref[...], (((1,), (0,)), ((), ())),
                         preferred_element_type=jnp.float32) + b2_ref[...][None, :]
    out_ref[0] = feat + h2


def _full(shape):
    return pl.BlockSpec(shape, lambda b: (0,) * len(shape))


def _batched(shape):
    return pl.BlockSpec((1,) + shape, lambda b: (b,) + (0,) * len(shape))


@jax.jit
def kernel(x, norm1_w, norm1_b, score_w, score_b, Wq, bq, Wkv, bkv, Wproj,
           bproj, norm2_w, norm2_b, W1, b1, W2, b2):
    mu = jnp.mean(x, axis=-1, keepdims=True)
    var = jnp.mean((x - mu) ** 2, axis=-1, keepdims=True)
    xn = (x - mu) / jnp.sqrt(var + 1e-5) * norm1_w + norm1_b
    sq = jnp.sum(xn * xn, axis=-1)
    sq_row = sq.reshape(B, N, 1)
    sq_col = sq.reshape(B, 1, N)

    near32, dmax, ts, tw = pl.pallas_call(
        _near_kernel,
        grid=(B,),
        in_specs=[
            _batched((N, DIM)),
            _batched((N, 1)),
            _batched((N, 1)),
            _full((DIM,)), _full((DIM,)),
            _batched((N, 1)),
            _batched((1, N)),
            _full((DIM, 1)), _full((1,)),
        ],
        out_specs=[
            _batched((N, K)),
            _batched((1, 128)),
            _batched((1, N)),
            _batched((1, N)),
        ],
        out_shape=[
            jax.ShapeDtypeStruct((B, N, K), jnp.float32),
            jax.ShapeDtypeStruct((B, 1, 128), jnp.float32),
            jax.ShapeDtypeStruct((B, 1, N), jnp.float32),
            jax.ShapeDtypeStruct((B, 1, N), jnp.float32),
        ],
        scratch_shapes=[pltpu.VMEM((N, DIM), jnp.float32)],
    )(x, mu, var, norm1_w, norm1_b, sq_row, sq_col, score_w, score_b)

    density = jnp.exp(-jnp.mean(near32 ** 2, axis=-1))
    noise = jax.random.uniform(jax.random.key(42), density.shape,
                               dtype=density.dtype) * 1e-6
    density = density + noise
    dens_row = density.reshape(B, N, 1)
    dens_col = density.reshape(B, 1, N)

    idx = pl.pallas_call(
        _assign_kernel,
        grid=(B,),
        in_specs=[
            _batched((N, DIM)),
            _batched((N, 1)),
            _batched((N, 1)),
            _full((DIM,)), _full((DIM,)),
            _batched((N, 1)),
            _batched((1, N)),
            _batched((N, 1)),
            _batched((1, N)),
            _batched((1, 128)),
        ],
        out_specs=_batched((1, N)),
        out_shape=jax.ShapeDtypeStruct((B, 1, N), jnp.int32),
        scratch_shapes=[
            pltpu.VMEM((N, DIM), jnp.float32),
            pltpu.VMEM((N, N), jnp.float32),
            pltpu.VMEM((N, 1), jnp.float32),
            pltpu.VMEM((N, 1), jnp.int32),
        ],
    )(x, mu, var, norm1_w, norm1_b, sq_row, sq_col, dens_row, dens_col, dmax)

    out = pl.pallas_call(
        _block_kernel,
        grid=(B,),
        in_specs=[
            _batched((N, DIM)),
            _batched((N, 1)),
            _batched((N, 1)),
            _full((DIM,)), _full((DIM,)),
            _batched((1, N)),
            _batched((1, N)),
            _batched((1, N)),
            _full((DIM, DIM)), _full((DIM,)),
            _full((DIM, 2 * DIM)), _full((2 * DIM,)),
            _full((DIM, DIM)), _full((DIM,)),
            _full((DIM,)), _full((DIM,)),
            _full((DIM, MLP_HIDDEN)), _full((MLP_HIDDEN,)),
            _full((MLP_HIDDEN, DIM)), _full((DIM,)),
        ],
        out_specs=_batched((S, DIM)),
        out_shape=jax.ShapeDtypeStruct((B, S, DIM), jnp.float32),
    )(x, mu, var, norm1_w, norm1_b, idx, ts, tw, Wq, bq, Wkv, bkv, Wproj, bproj,
      norm2_w, norm2_b, W1, b1, W2, b2)
    return out
